# baseline (speedup 1.0000x reference)
import functools

import jax
import jax.numpy as jnp
from jax import lax
from jax.experimental import pallas as pl
from jax.experimental.pallas import tpu as pltpu
from jax.experimental.pallas import tpu_sc as plsc

N = 10000
K = 32
D = 128
NE = N * K
NUM_LAYERS = 3

_NC, _NS = 2, 16
_NW = _NC * _NS
_CH = 128

_SPLITS = (2400, 2400, 2600, 2600)


def _sc_gather(table, idx, per_w):
    full = per_w // _CH
    tail = per_w - full * _CH
    assert full % 2 == 0 and full >= 2 and tail % 8 == 0
    mesh = plsc.VectorSubcoreMesh(core_axis_name="c", subcore_axis_name="s")

    scratch = [
        pltpu.VMEM((per_w,), jnp.int32),
        pltpu.VMEM((_CH, D), jnp.float32),
        pltpu.VMEM((_CH, D), jnp.float32),
        pltpu.VMEM((max(tail, 8), D), jnp.float32),
        pltpu.SemaphoreType.DMA,
        pltpu.SemaphoreType.DMA,
        pltpu.SemaphoreType.DMA,
    ]

    @functools.partial(
        pl.kernel,
        out_type=jax.ShapeDtypeStruct((per_w * _NW, D), jnp.float32),
        mesh=mesh,
        scratch_types=scratch,
    )
    def gather_kernel(table_hbm, idx_hbm, out_hbm, idx_all,
                      rows0, rows1, rows_t, sem0, sem1, sem_t):
        wid = lax.axis_index("s") * _NC + lax.axis_index("c")
        base = wid * per_w

        def fire(c, rowsb, semb):
            pltpu.async_copy(
                table_hbm.at[idx_all.at[pl.ds(c * _CH, _CH)]], rowsb, semb)

        def wait_wb(c, rowsb, semb):
            pltpu.make_async_copy(
                table_hbm.at[pl.ds(0, _CH)], rowsb, semb).wait()
            pltpu.sync_copy(rowsb, out_hbm.at[pl.ds(base + c * _CH, _CH)])

        pltpu.sync_copy(idx_hbm.at[pl.ds(base, per_w)], idx_all)
        if tail:
            pltpu.async_copy(
                table_hbm.at[idx_all.at[pl.ds(full * _CH, tail)]],
                rows_t.at[pl.ds(0, tail)], sem_t)
        fire(0, rows0, sem0)
        fire(1, rows1, sem1)

        @pl.loop(0, full - 2, step=2)
        def _(i):
            wait_wb(i, rows0, sem0)
            fire(i + 2, rows0, sem0)
            wait_wb(i + 1, rows1, sem1)
            fire(i + 3, rows1, sem1)

        wait_wb(full - 2, rows0, sem0)
        wait_wb(full - 1, rows1, sem1)
        if tail:
            pltpu.make_async_copy(
                table_hbm.at[pl.ds(0, tail)], rows_t.at[pl.ds(0, tail)],
                sem_t).wait()
            pltpu.sync_copy(rows_t.at[pl.ds(0, tail)],
                            out_hbm.at[pl.ds(base + full * _CH, tail)])

    return gather_kernel(table, idx)


_B = 200
_F32 = jnp.float32


def _gelu(x):
    return 0.5 * x * (1.0 + lax.erf(x * 0.7071067811865476))


def _ln(x, g, b):
    mu = jnp.mean(x, axis=-1, keepdims=True)
    xc = x - mu
    var = jnp.mean(xc * xc, axis=-1, keepdims=True)
    return xc * lax.rsqrt(var + 1e-5) * g + b


def _decoder_body(h_ref, e_ref, g_ref, m_ref,
                  w1h, w1eg, b1, w2, b2, w3, b3,
                  wd1, bd1, wd2, bd2, g1, n1b, g2, n2b,
                  out_ref):
    bf16 = jnp.bfloat16
    h = h_ref[...]
    eg = jnp.concatenate(
        [e_ref[...].reshape(_B * K, D).astype(bf16),
         g_ref[...].astype(bf16)], axis=-1)
    msk = m_ref[...]
    et_all = jnp.dot(eg, w1eg[...],
                     preferred_element_type=_F32).astype(bf16)
    for l in range(NUM_LAYERS):
        et = et_all[:, l * D:(l + 1) * D]
        ht = (jnp.dot(h, w1h[l], preferred_element_type=_F32)
              + b1[l]).astype(bf16)
        m1 = _gelu(et.reshape(_B, K, D) + ht[:, None, :]).reshape(_B * K, D)
        m2 = _gelu(jnp.dot(m1, w2[l],
                           preferred_element_type=_F32).astype(bf16) + b2[l])
        m3 = jnp.dot(m2, w3[l], preferred_element_type=_F32)
        agg = jnp.sum(m3.reshape(_B, K, D), axis=1)
        h = _ln(h + agg * (1.0 / 30.0) + b3[l] * (K / 30.0), g1[l], n1b[l])
        d = _gelu(jnp.dot(h, wd1[l], preferred_element_type=_F32) + bd1[l])
        h = _ln(h + jnp.dot(d, wd2[l], preferred_element_type=_F32) + bd2[l],
                g2[l], n2b[l])
        h = h * msk
    out_ref[...] = h


def _full(shape):
    nd = len(shape)
    return pl.BlockSpec(shape, lambda i: (0,) * nd)


def _tc_decoder(h0, edge, gath, mask2, weights, nb, off_b):
    in_specs = [
        pl.BlockSpec((_B, D), lambda i: (i + off_b, 0)),
        pl.BlockSpec((_B, K, D), lambda i: (i + off_b, 0, 0)),
        pl.BlockSpec((_B * K, D), lambda i: (i, 0)),
        pl.BlockSpec((_B, 1), lambda i: (i + off_b, 0)),
    ] + [_full(w.shape) for w in weights]
    return pl.pallas_call(
        _decoder_body,
        grid=(nb,),
        in_specs=in_specs,
        out_specs=pl.BlockSpec((_B, D), lambda i: (i, 0)),
        out_shape=jax.ShapeDtypeStruct((nb * _B, D), _F32),
        compiler_params=pltpu.CompilerParams(
            dimension_semantics=("arbitrary",)),
    )(h0, edge, gath, mask2, *weights)


def kernel(node_features, edge_features, neighbor_indices, mask, params):
    idx = neighbor_indices.reshape(-1).astype(jnp.int32)

    def st(fn):
        return jnp.stack([fn(p) for p in params])

    w1eg = jnp.concatenate(
        [jnp.concatenate([p["message"][0]["W"][:, D:2 * D].T,
                          p["message"][0]["W"][:, 3 * D:4 * D].T], axis=0)
         for p in params], axis=1).astype(jnp.bfloat16)
    weights = [
        st(lambda p: p["message"][0]["W"][:, :D].T),
        w1eg,
        st(lambda p: p["message"][0]["b"][None, :]),
        st(lambda p: p["message"][1]["W"].T.astype(jnp.bfloat16)),
        st(lambda p: p["message"][1]["b"][None, :].astype(jnp.bfloat16)),
        st(lambda p: p["message"][2]["W"].T.astype(jnp.bfloat16)),
        st(lambda p: p["message"][2]["b"][None, :]),
        st(lambda p: p["dense"][0]["W"].T),
        st(lambda p: p["dense"][0]["b"][None, :]),
        st(lambda p: p["dense"][1]["W"].T),
        st(lambda p: p["dense"][1]["b"][None, :]),
        st(lambda p: p["norm1"]["g"][None, :]),
        st(lambda p: p["norm1"]["b"][None, :]),
        st(lambda p: p["norm2"]["g"][None, :]),
        st(lambda p: p["norm2"]["b"][None, :]),
    ]
    mask2 = mask[:, None]
    gathers, off = [], 0
    for s in _SPLITS:
        gathers.append(_sc_gather(node_features, idx[off * K:(off + s) * K], s))
        off += s
    outs, off = [], 0
    for s, g in zip(_SPLITS, gathers):
        outs.append(_tc_decoder(node_features, edge_features, g, mask2,
                                weights, s // _B, off // _B))
        off += s
    return jnp.concatenate(outs, axis=0)


# speedup vs baseline: 1.1085x; 1.1085x over previous
import functools

import jax
import jax.numpy as jnp
from jax import lax
from jax.experimental import pallas as pl
from jax.experimental.pallas import tpu as pltpu
from jax.experimental.pallas import tpu_sc as plsc

N = 10000
K = 32
D = 128
NE = N * K
NUM_LAYERS = 3

_NC, _NS = 2, 16
_NW = _NC * _NS
_CH = 128

_SPLITS = (1600, 2400, 2400, 3600)


def _sc_gather(table, idx, per_w):
    full = per_w // _CH
    tail = per_w - full * _CH
    assert full % 2 == 0 and full >= 2 and tail % 8 == 0
    mesh = plsc.VectorSubcoreMesh(core_axis_name="c", subcore_axis_name="s")

    scratch = [
        pltpu.VMEM((per_w,), jnp.int32),
        pltpu.VMEM((_CH, D), jnp.float32),
        pltpu.VMEM((_CH, D), jnp.float32),
        pltpu.VMEM((max(tail, 8), D), jnp.float32),
        pltpu.SemaphoreType.DMA,
        pltpu.SemaphoreType.DMA,
        pltpu.SemaphoreType.DMA,
    ]

    @functools.partial(
        pl.kernel,
        out_type=jax.ShapeDtypeStruct((per_w * _NW, D), jnp.float32),
        mesh=mesh,
        scratch_types=scratch,
    )
    def gather_kernel(table_hbm, idx_hbm, out_hbm, idx_all,
                      rows0, rows1, rows_t, sem0, sem1, sem_t):
        wid = lax.axis_index("s") * _NC + lax.axis_index("c")
        base = wid * per_w

        def fire(c, rowsb, semb):
            pltpu.async_copy(
                table_hbm.at[idx_all.at[pl.ds(c * _CH, _CH)]], rowsb, semb)

        def wait_wb(c, rowsb, semb):
            pltpu.make_async_copy(
                table_hbm.at[pl.ds(0, _CH)], rowsb, semb).wait()
            pltpu.sync_copy(rowsb, out_hbm.at[pl.ds(base + c * _CH, _CH)])

        pltpu.sync_copy(idx_hbm.at[pl.ds(base, per_w)], idx_all)
        if tail:
            pltpu.async_copy(
                table_hbm.at[idx_all.at[pl.ds(full * _CH, tail)]],
                rows_t.at[pl.ds(0, tail)], sem_t)
        fire(0, rows0, sem0)
        fire(1, rows1, sem1)

        @pl.loop(0, full - 2, step=2)
        def _(i):
            wait_wb(i, rows0, sem0)
            fire(i + 2, rows0, sem0)
            wait_wb(i + 1, rows1, sem1)
            fire(i + 3, rows1, sem1)

        wait_wb(full - 2, rows0, sem0)
        wait_wb(full - 1, rows1, sem1)
        if tail:
            pltpu.make_async_copy(
                table_hbm.at[pl.ds(0, tail)], rows_t.at[pl.ds(0, tail)],
                sem_t).wait()
            pltpu.sync_copy(rows_t.at[pl.ds(0, tail)],
                            out_hbm.at[pl.ds(base + full * _CH, tail)])

    return gather_kernel(table, idx)


_B = 400
_F32 = jnp.float32


def _gelu(x):
    return 0.5 * x * (1.0 + lax.erf(x * 0.7071067811865476))


def _ln(x, g, b):
    mu = jnp.mean(x, axis=-1, keepdims=True)
    xc = x - mu
    var = jnp.mean(xc * xc, axis=-1, keepdims=True)
    return xc * lax.rsqrt(var + 1e-5) * g + b


def _decoder_body(h_ref, e_ref, g_ref, m_ref,
                  w1h, w1eg, b1, w2, b2, w3, b3,
                  wd1, bd1, wd2, bd2, g1, n1b, g2, n2b,
                  out_ref):
    bf16 = jnp.bfloat16
    h = h_ref[...]
    eg = jnp.concatenate(
        [e_ref[...].reshape(_B * K, D).astype(bf16),
         g_ref[...].astype(bf16)], axis=-1)
    msk = m_ref[...]
    et_all = jnp.dot(eg, w1eg[...],
                     preferred_element_type=_F32).astype(bf16)
    for l in range(NUM_LAYERS):
        et = et_all[:, l * D:(l + 1) * D]
        ht = (jnp.dot(h, w1h[l], preferred_element_type=_F32)
              + b1[l]).astype(bf16)
        m1 = _gelu(et.reshape(_B, K, D) + ht[:, None, :]).reshape(_B * K, D)
        m2 = _gelu(jnp.dot(m1, w2[l],
                           preferred_element_type=_F32).astype(bf16) + b2[l])
        m3 = jnp.dot(m2, w3[l], preferred_element_type=_F32)
        agg = jnp.sum(m3.reshape(_B, K, D), axis=1)
        h = _ln(h + agg * (1.0 / 30.0) + b3[l] * (K / 30.0), g1[l], n1b[l])
        d = _gelu(jnp.dot(h, wd1[l], preferred_element_type=_F32) + bd1[l])
        h = _ln(h + jnp.dot(d, wd2[l], preferred_element_type=_F32) + bd2[l],
                g2[l], n2b[l])
        h = h * msk
    out_ref[...] = h


def _full(shape):
    nd = len(shape)
    return pl.BlockSpec(shape, lambda i: (0,) * nd)


def _tc_decoder(h0, edge, gath, mask2, weights, nb, off_b):
    in_specs = [
        pl.BlockSpec((_B, D), lambda i: (i + off_b, 0)),
        pl.BlockSpec((_B, K, D), lambda i: (i + off_b, 0, 0)),
        pl.BlockSpec((_B * K, D), lambda i: (i, 0)),
        pl.BlockSpec((_B, 1), lambda i: (i + off_b, 0)),
    ] + [_full(w.shape) for w in weights]
    return pl.pallas_call(
        _decoder_body,
        grid=(nb,),
        in_specs=in_specs,
        out_specs=pl.BlockSpec((_B, D), lambda i: (i, 0)),
        out_shape=jax.ShapeDtypeStruct((nb * _B, D), _F32),
        compiler_params=pltpu.CompilerParams(
            dimension_semantics=("arbitrary",)),
    )(h0, edge, gath, mask2, *weights)


def kernel(node_features, edge_features, neighbor_indices, mask, params):
    idx = neighbor_indices.reshape(-1).astype(jnp.int32)

    def st(fn):
        return jnp.stack([fn(p) for p in params])

    w1eg = jnp.concatenate(
        [jnp.concatenate([p["message"][0]["W"][:, D:2 * D].T,
                          p["message"][0]["W"][:, 3 * D:4 * D].T], axis=0)
         for p in params], axis=1).astype(jnp.bfloat16)
    weights = [
        st(lambda p: p["message"][0]["W"][:, :D].T),
        w1eg,
        st(lambda p: p["message"][0]["b"][None, :]),
        st(lambda p: p["message"][1]["W"].T.astype(jnp.bfloat16)),
        st(lambda p: p["message"][1]["b"][None, :].astype(jnp.bfloat16)),
        st(lambda p: p["message"][2]["W"].T.astype(jnp.bfloat16)),
        st(lambda p: p["message"][2]["b"][None, :]),
        st(lambda p: p["dense"][0]["W"].T),
        st(lambda p: p["dense"][0]["b"][None, :]),
        st(lambda p: p["dense"][1]["W"].T),
        st(lambda p: p["dense"][1]["b"][None, :]),
        st(lambda p: p["norm1"]["g"][None, :]),
        st(lambda p: p["norm1"]["b"][None, :]),
        st(lambda p: p["norm2"]["g"][None, :]),
        st(lambda p: p["norm2"]["b"][None, :]),
    ]
    mask2 = mask[:, None]
    gathers, off = [], 0
    for s in _SPLITS:
        gathers.append(_sc_gather(node_features, idx[off * K:(off + s) * K], s))
        off += s
    outs, off = [], 0
    for s, g in zip(_SPLITS, gathers):
        outs.append(_tc_decoder(node_features, edge_features, g, mask2,
                                weights, s // _B, off // _B))
        off += s
    return jnp.concatenate(outs, axis=0)


# speedup vs baseline: 1.1660x; 1.0519x over previous
import functools

import jax
import jax.numpy as jnp
from jax import lax
from jax.experimental import pallas as pl
from jax.experimental.pallas import tpu as pltpu
from jax.experimental.pallas import tpu_sc as plsc

N = 10000
K = 32
D = 128
NE = N * K
NUM_LAYERS = 3

_NC, _NS = 2, 16
_NW = _NC * _NS
_CH = 128

_SPLITS = (1600, 2400, 6000)


def _sc_gather(table, idx, per_w):
    full = per_w // _CH
    tail = per_w - full * _CH
    assert full % 2 == 0 and full >= 2 and tail % 8 == 0
    mesh = plsc.VectorSubcoreMesh(core_axis_name="c", subcore_axis_name="s")

    scratch = [
        pltpu.VMEM((per_w,), jnp.int32),
        pltpu.VMEM((_CH, D), jnp.float32),
        pltpu.VMEM((_CH, D), jnp.float32),
        pltpu.VMEM((max(tail, 8), D), jnp.float32),
        pltpu.SemaphoreType.DMA,
        pltpu.SemaphoreType.DMA,
        pltpu.SemaphoreType.DMA,
    ]

    @functools.partial(
        pl.kernel,
        out_type=jax.ShapeDtypeStruct((per_w * _NW, D), jnp.float32),
        mesh=mesh,
        scratch_types=scratch,
    )
    def gather_kernel(table_hbm, idx_hbm, out_hbm, idx_all,
                      rows0, rows1, rows_t, sem0, sem1, sem_t):
        wid = lax.axis_index("s") * _NC + lax.axis_index("c")
        base = wid * per_w

        def fire(c, rowsb, semb):
            pltpu.async_copy(
                table_hbm.at[idx_all.at[pl.ds(c * _CH, _CH)]], rowsb, semb)

        def wait_wb(c, rowsb, semb):
            pltpu.make_async_copy(
                table_hbm.at[pl.ds(0, _CH)], rowsb, semb).wait()
            pltpu.sync_copy(rowsb, out_hbm.at[pl.ds(base + c * _CH, _CH)])

        pltpu.sync_copy(idx_hbm.at[pl.ds(base, per_w)], idx_all)
        if tail:
            pltpu.async_copy(
                table_hbm.at[idx_all.at[pl.ds(full * _CH, tail)]],
                rows_t.at[pl.ds(0, tail)], sem_t)
        fire(0, rows0, sem0)
        fire(1, rows1, sem1)

        @pl.loop(0, full - 2, step=2)
        def _(i):
            wait_wb(i, rows0, sem0)
            fire(i + 2, rows0, sem0)
            wait_wb(i + 1, rows1, sem1)
            fire(i + 3, rows1, sem1)

        wait_wb(full - 2, rows0, sem0)
        wait_wb(full - 1, rows1, sem1)
        if tail:
            pltpu.make_async_copy(
                table_hbm.at[pl.ds(0, tail)], rows_t.at[pl.ds(0, tail)],
                sem_t).wait()
            pltpu.sync_copy(rows_t.at[pl.ds(0, tail)],
                            out_hbm.at[pl.ds(base + full * _CH, tail)])

    return gather_kernel(table, idx)


_B = 400
_F32 = jnp.float32


def _gelu(x):
    return 0.5 * x * (1.0 + lax.erf(x * 0.7071067811865476))


def _ln(x, g, b):
    mu = jnp.mean(x, axis=-1, keepdims=True)
    xc = x - mu
    var = jnp.mean(xc * xc, axis=-1, keepdims=True)
    return xc * lax.rsqrt(var + 1e-5) * g + b


def _decoder_body(h_ref, e_ref, g_ref, m_ref,
                  w1h, w1eg, b1, w2, b2, w3, b3,
                  wd1, bd1, wd2, bd2, g1, n1b, g2, n2b,
                  out_ref):
    bf16 = jnp.bfloat16
    h = h_ref[...]
    eg = jnp.concatenate(
        [e_ref[...].reshape(_B * K, D).astype(bf16),
         g_ref[...].astype(bf16)], axis=-1)
    msk = m_ref[...]
    et_all = jnp.dot(eg, w1eg[...],
                     preferred_element_type=_F32).astype(bf16)
    for l in range(NUM_LAYERS):
        et = et_all[:, l * D:(l + 1) * D]
        ht = (jnp.dot(h, w1h[l], preferred_element_type=_F32)
              + b1[l]).astype(bf16)
        m1 = _gelu(et.reshape(_B, K, D) + ht[:, None, :]).reshape(_B * K, D)
        m2 = _gelu(jnp.dot(m1, w2[l],
                           preferred_element_type=_F32).astype(bf16) + b2[l])
        m3 = jnp.dot(m2, w3[l], preferred_element_type=_F32)
        agg = jnp.sum(m3.reshape(_B, K, D), axis=1)
        h = _ln(h + agg * (1.0 / 30.0) + b3[l] * (K / 30.0), g1[l], n1b[l])
        d = _gelu(jnp.dot(h, wd1[l], preferred_element_type=_F32) + bd1[l])
        h = _ln(h + jnp.dot(d, wd2[l], preferred_element_type=_F32) + bd2[l],
                g2[l], n2b[l])
        h = h * msk
    out_ref[...] = h


def _full(shape):
    nd = len(shape)
    return pl.BlockSpec(shape, lambda i: (0,) * nd)


def _tc_decoder(h0, edge, gath, mask2, weights, nb, off_b):
    in_specs = [
        pl.BlockSpec((_B, D), lambda i: (i + off_b, 0)),
        pl.BlockSpec((_B, K, D), lambda i: (i + off_b, 0, 0)),
        pl.BlockSpec((_B * K, D), lambda i: (i, 0)),
        pl.BlockSpec((_B, 1), lambda i: (i + off_b, 0)),
    ] + [_full(w.shape) for w in weights]
    return pl.pallas_call(
        _decoder_body,
        grid=(nb,),
        in_specs=in_specs,
        out_specs=pl.BlockSpec((_B, D), lambda i: (i, 0)),
        out_shape=jax.ShapeDtypeStruct((nb * _B, D), _F32),
        compiler_params=pltpu.CompilerParams(
            dimension_semantics=("arbitrary",)),
    )(h0, edge, gath, mask2, *weights)


def kernel(node_features, edge_features, neighbor_indices, mask, params):
    idx = neighbor_indices.reshape(-1).astype(jnp.int32)

    def st(fn):
        return jnp.stack([fn(p) for p in params])

    w1eg = jnp.concatenate(
        [jnp.concatenate([p["message"][0]["W"][:, D:2 * D].T,
                          p["message"][0]["W"][:, 3 * D:4 * D].T], axis=0)
         for p in params], axis=1).astype(jnp.bfloat16)
    weights = [
        st(lambda p: p["message"][0]["W"][:, :D].T),
        w1eg,
        st(lambda p: p["message"][0]["b"][None, :]),
        st(lambda p: p["message"][1]["W"].T.astype(jnp.bfloat16)),
        st(lambda p: p["message"][1]["b"][None, :].astype(jnp.bfloat16)),
        st(lambda p: p["message"][2]["W"].T.astype(jnp.bfloat16)),
        st(lambda p: p["message"][2]["b"][None, :]),
        st(lambda p: p["dense"][0]["W"].T),
        st(lambda p: p["dense"][0]["b"][None, :]),
        st(lambda p: p["dense"][1]["W"].T),
        st(lambda p: p["dense"][1]["b"][None, :]),
        st(lambda p: p["norm1"]["g"][None, :]),
        st(lambda p: p["norm1"]["b"][None, :]),
        st(lambda p: p["norm2"]["g"][None, :]),
        st(lambda p: p["norm2"]["b"][None, :]),
    ]
    mask2 = mask[:, None]
    gathers, off = [], 0
    for s in _SPLITS:
        gathers.append(_sc_gather(node_features, idx[off * K:(off + s) * K], s))
        off += s
    outs, off = [], 0
    for s, g in zip(_SPLITS, gathers):
        outs.append(_tc_decoder(node_features, edge_features, g, mask2,
                                weights, s // _B, off // _B))
        off += s
    return jnp.concatenate(outs, axis=0)
